# Initial kernel scaffold; baseline (speedup 1.0000x reference)
import jax
import jax.numpy as jnp
from jax.experimental import pallas as pl


def kernel(x, pos, batch, W1, b1, gamma, beta, W2, b2, t):
    raise NotImplementedError("write your pallas kernel here")


# speedup vs baseline: 4.2074x; 4.2074x over previous
import jax
import jax.numpy as jnp
from jax import lax
from jax.experimental import pallas as pl
from jax.experimental.pallas import tpu as pltpu

B = 8
NPG = 1024
K = 16
CIN = 256
COUT = 10
CPAD = 16
N = B * NPG
NEG = -3.0e38


def _mlp_body(x_ref, w1_ref, b1_ref, gamma_ref, beta_ref, w2_ref, bn_ref, emb_ref):
    x = x_ref[...]
    h = jnp.dot(x, w1_ref[...], preferred_element_type=jnp.float32) + b1_ref[...]
    mu = jnp.mean(h, axis=0, keepdims=True)
    var = jnp.mean((h - mu) ** 2, axis=0, keepdims=True)
    h = (h - mu) / jnp.sqrt(var + 1e-5) * gamma_ref[...] + beta_ref[...]
    h = jnp.maximum(h, 0.0)
    emb_ref[...] = jnp.dot(h, w2_ref[...], preferred_element_type=jnp.float32) + bn_ref[...]


def _topk16(score):
    ci = lax.broadcasted_iota(jnp.int32, (NPG, NPG), 1)
    vals, idxs = [], []
    s = score
    for _ in range(K):
        m = jnp.max(s, axis=1, keepdims=True)
        j = jnp.min(jnp.where(s == m, ci, NPG), axis=1, keepdims=True)
        vals.append(m)
        idxs.append(j)
        s = jnp.where(ci == j, NEG, s)
    return jnp.concatenate(vals, axis=1), jnp.concatenate(idxs, axis=1)


def _scores_body(t_ref, pos_ref, post_ref, emb_ref, embt_ref, gt_ref,
                 knn_idx_ref, gum_idx_ref, gum_val_ref):
    b = pl.program_id(0)
    off = (b * NPG).astype(jnp.int32)
    ri = lax.broadcasted_iota(jnp.int32, (NPG, NPG), 0)
    ci = lax.broadcasted_iota(jnp.int32, (NPG, NPG), 1)

    pos = pos_ref[0]
    post = post_ref[0]
    s_row = jnp.sum(pos * pos, axis=1, keepdims=True)
    s_col = jnp.sum(post * post, axis=0, keepdims=True)
    gram = jnp.dot(pos, post, preferred_element_type=jnp.float32)
    d2 = jnp.maximum(s_row + s_col - 2.0 * gram, 0.0)
    d2 = jnp.where(ri == ci, d2 + 1e12, d2)
    _, kidx = _topk16(-d2)
    knn_idx_ref[0] = kidx + off

    emb = emb_ref[0]
    embt = embt_ref[0]
    e_row = jnp.sum(emb * emb, axis=1, keepdims=True)
    e_col = jnp.sum(embt * embt, axis=0, keepdims=True)
    egram = jnp.dot(emb, embt, preferred_element_type=jnp.float32)
    ed2 = jnp.maximum(e_row + e_col - 2.0 * egram, 0.0)
    dist = jnp.sqrt(ed2 + 1e-12)
    t = t_ref[0, 0]
    p = jnp.exp(-t * dist * dist)
    noisy_t = jnp.log(p + 1e-20) + gt_ref[0]
    gvals, gidx = _topk16(noisy_t)
    e = jnp.exp(gvals - gvals[:, 0:1])
    sm = e / jnp.sum(e, axis=1, keepdims=True)
    gum_val_ref[0] = sm / jnp.max(sm, axis=1, keepdims=True)
    gum_idx_ref[0] = gidx + off


def kernel(x, pos, batch, W1, b1, gamma, beta, W2, b2, t):
    f32 = jnp.float32
    nz = jax.random.uniform(jax.random.key(1), (N, COUT), dtype=f32) * 0.001
    u = jax.random.uniform(jax.random.key(2), (B, NPG, NPG), dtype=f32)
    gt = -jnp.log(-jnp.log(jnp.swapaxes(u, 1, 2) + 1e-20) + 1e-20)
    w2p = jnp.pad(W2, ((0, 0), (0, CPAD - COUT)))
    bn = jnp.pad(b2[None, :] + nz, ((0, 0), (0, CPAD - COUT)))

    emb = pl.pallas_call(
        _mlp_body,
        out_shape=jax.ShapeDtypeStruct((N, CPAD), f32),
    )(x, W1, b1[None, :], gamma[None, :], beta[None, :], w2p, bn)

    posp = jnp.pad(pos, ((0, 0), (0, 5))).reshape(B, NPG, 8)
    post = jnp.swapaxes(posp, 1, 2)
    embr = emb.reshape(B, NPG, CPAD)
    embt = jnp.swapaxes(embr, 1, 2)
    t2 = t.reshape(1, 1)

    g3 = lambda i: (i, 0, 0)
    knn_idx, gum_idx, gum_val = pl.pallas_call(
        _scores_body,
        grid=(B,),
        in_specs=[
            pl.BlockSpec((1, 1), lambda i: (0, 0)),
            pl.BlockSpec((1, NPG, 8), g3),
            pl.BlockSpec((1, 8, NPG), g3),
            pl.BlockSpec((1, NPG, CPAD), g3),
            pl.BlockSpec((1, CPAD, NPG), g3),
            pl.BlockSpec((1, NPG, NPG), g3),
        ],
        out_specs=[
            pl.BlockSpec((1, NPG, K), g3),
            pl.BlockSpec((1, NPG, K), g3),
            pl.BlockSpec((1, NPG, K), g3),
        ],
        out_shape=[
            jax.ShapeDtypeStruct((B, NPG, K), jnp.int32),
            jax.ShapeDtypeStruct((B, NPG, K), jnp.int32),
            jax.ShapeDtypeStruct((B, NPG, K), f32),
        ],
    )(t2, posp, post, embr, embt, gt)

    rows = jnp.repeat(jnp.arange(N, dtype=jnp.int32), K)
    knn_edge = jnp.stack([knn_idx.reshape(-1), rows], axis=0)
    soft_index_i = jnp.stack([gum_idx.reshape(-1), rows], axis=0)
    soft_index_v = jnp.stack([gum_val.reshape(-1), rows.astype(f32)], axis=0)
    edge_index = jnp.concatenate([soft_index_i, knn_edge], axis=1)
    return edge_index, soft_index_i, soft_index_v
